# Initial kernel scaffold; baseline (speedup 1.0000x reference)
import functools

import jax
import jax.numpy as jnp
from jax import lax
from jax.experimental import pallas as pl
from jax.experimental.pallas import tpu as pltpu, tpu_sc as plsc

N = 10000
D = 256
H = 16
E = 160000
NP = 10240
NT = 32
CHUNK = 128
NCHUNK = 40
EPT = CHUNK * NCHUNK
EP = NT * EPT
STRIPE = NP // 16

_mesh = plsc.VectorSubcoreMesh(core_axis_name="c", subcore_axis_name="s")


@functools.partial(
    pl.kernel,
    out_type=jax.ShapeDtypeStruct((2 * NP,), jnp.float32),
    mesh=_mesh,
    scratch_types=[
        pltpu.VMEM((NCHUNK, CHUNK), jnp.int32),
        pltpu.VMEM((CHUNK,), jnp.float32),
        pltpu.VMEM_SHARED((NP,), jnp.float32),
    ],
)
def _sc_deg(dst_hbm, z1_hbm, deg_out, idx_v, ones_v, deg_s):
    cid = lax.axis_index("c")
    sid = lax.axis_index("s")
    wid = sid * 2 + cid
    pltpu.sync_copy(dst_hbm.at[wid], idx_v)
    for k in range(CHUNK // 16):
        ones_v[pl.ds(k * 16, 16)] = jnp.ones((16,), jnp.float32)
    pltpu.sync_copy(z1_hbm, deg_s.at[pl.ds(sid * STRIPE, STRIPE)])
    plsc.subcore_barrier()

    def body(j, carry):
        pltpu.sync_copy(ones_v, deg_s.at[idx_v.at[j]], add=True)
        return carry

    lax.fori_loop(0, NCHUNK, body, 0)
    plsc.subcore_barrier()
    pltpu.sync_copy(deg_s.at[pl.ds(sid * STRIPE, STRIPE)],
                    deg_out.at[pl.ds(cid * NP + sid * STRIPE, STRIPE)])


_ROWS = 1024


def _tc1_body(x_ref, w1_ref, d0_ref, d1_ref, g0_ref, dinv_ref):
    h0 = jnp.dot(x_ref[...], w1_ref[...], preferred_element_type=jnp.float32)
    deg = d0_ref[...] + d1_ref[...] + 1.0
    dinv = lax.rsqrt(deg)
    dinv_ref[...] = dinv
    g0_ref[...] = h0 * dinv


def _tc1(xp, W1, deg0, deg1):
    return pl.pallas_call(
        _tc1_body,
        grid=(NP // _ROWS,),
        in_specs=[
            pl.BlockSpec((_ROWS, D), lambda i: (i, 0)),
            pl.BlockSpec((D, H), lambda i: (0, 0)),
            pl.BlockSpec((_ROWS, 1), lambda i: (i, 0)),
            pl.BlockSpec((_ROWS, 1), lambda i: (i, 0)),
        ],
        out_specs=[
            pl.BlockSpec((_ROWS, H), lambda i: (i, 0)),
            pl.BlockSpec((_ROWS, 1), lambda i: (i, 0)),
        ],
        out_shape=[
            jax.ShapeDtypeStruct((NP, H), jnp.float32),
            jax.ShapeDtypeStruct((NP, 1), jnp.float32),
        ],
    )(xp, W1, deg0, deg1)


@functools.partial(
    pl.kernel,
    out_type=[
        jax.ShapeDtypeStruct((2 * NP, H), jnp.float32),
        jax.ShapeDtypeStruct((2 * NP,), jnp.float32),
    ],
    mesh=_mesh,
    scratch_types=[
        pltpu.VMEM((NCHUNK, CHUNK), jnp.int32),
        pltpu.VMEM((NCHUNK, CHUNK), jnp.int32),
        pltpu.VMEM((NP,), jnp.float32),
        pltpu.VMEM((CHUNK, H), jnp.float32),
        pltpu.VMEM((CHUNK,), jnp.float32),
        pltpu.VMEM_SHARED((NP, H), jnp.float32),
        pltpu.VMEM_SHARED((NP,), jnp.float32),
        pltpu.SemaphoreType.DMA,
    ],
)
def _sc_edge(g0_hbm, dinv_hbm, src_hbm, dst_hbm, zH_hbm, z1_hbm,
             acc_out, t_out,
             src_v, dst_v, dinv_v, rows_v, tvals_v, acc_s, t_s, sem):
    cid = lax.axis_index("c")
    sid = lax.axis_index("s")
    wid = sid * 2 + cid
    pltpu.sync_copy(src_hbm.at[wid], src_v)
    pltpu.sync_copy(dst_hbm.at[wid], dst_v)
    pltpu.sync_copy(dinv_hbm, dinv_v)
    pltpu.sync_copy(zH_hbm, acc_s.at[pl.ds(sid * STRIPE, STRIPE), :])
    pltpu.sync_copy(z1_hbm, t_s.at[pl.ds(sid * STRIPE, STRIPE)])
    plsc.subcore_barrier()

    def body(j, carry):
        pltpu.async_copy(g0_hbm.at[src_v.at[j]], rows_v, sem).wait()
        pltpu.sync_copy(rows_v, acc_s.at[dst_v.at[j]], add=True)
        for k in range(CHUNK // 16):
            dvec = plsc.load_gather(dinv_v, [dst_v[j, pl.ds(k * 16, 16)]])
            tvals_v[pl.ds(k * 16, 16)] = dvec
        pltpu.sync_copy(tvals_v, t_s.at[src_v.at[j]], add=True)
        return carry

    lax.fori_loop(0, NCHUNK, body, 0)
    plsc.subcore_barrier()
    pltpu.sync_copy(acc_s.at[pl.ds(sid * STRIPE, STRIPE), :],
                    acc_out.at[pl.ds(cid * NP + sid * STRIPE, STRIPE), :])
    pltpu.sync_copy(t_s.at[pl.ds(sid * STRIPE, STRIPE)],
                    t_out.at[pl.ds(cid * NP + sid * STRIPE, STRIPE)])


def _tc2_body(a0_ref, a1_ref, g0_ref, dinv_ref, t0_ref, t1_ref,
              b1_ref, w2_ref, b2_ref, w3_ref, b3_ref, out_ref, vacc):
    i = pl.program_id(0)
    dinv = dinv_ref[...]
    acc = a0_ref[...] + a1_ref[...] + g0_ref[...]
    h1 = jnp.maximum(dinv * acc + b1_ref[...], 0.0)
    w = dinv * (t0_ref[...] + t1_ref[...] + dinv)
    ridx = lax.broadcasted_iota(jnp.int32, (_ROWS, 1), 0) + i * _ROWS
    w = jnp.where(ridx < N, w, 0.0)
    part = jnp.sum(h1 * w, axis=0, keepdims=True)

    @pl.when(i == 0)
    def _():
        vacc[...] = part

    @pl.when(i > 0)
    def _():
        vacc[...] = vacc[...] + part

    @pl.when(i == NP // _ROWS - 1)
    def _():
        v = vacc[...] * (1.0 / N)
        pooled = jnp.dot(v, w2_ref[...], preferred_element_type=jnp.float32) + b2_ref[...]
        out_ref[...] = jnp.dot(pooled, w3_ref[...], preferred_element_type=jnp.float32) + b3_ref[...]


def _tc2(acc0, acc1, g0, dinv, t0, t1, b1, W2, b2, W3, b3):
    return pl.pallas_call(
        _tc2_body,
        grid=(NP // _ROWS,),
        in_specs=[
            pl.BlockSpec((_ROWS, H), lambda i: (i, 0)),
            pl.BlockSpec((_ROWS, H), lambda i: (i, 0)),
            pl.BlockSpec((_ROWS, H), lambda i: (i, 0)),
            pl.BlockSpec((_ROWS, 1), lambda i: (i, 0)),
            pl.BlockSpec((_ROWS, 1), lambda i: (i, 0)),
            pl.BlockSpec((_ROWS, 1), lambda i: (i, 0)),
            pl.BlockSpec((1, H), lambda i: (0, 0)),
            pl.BlockSpec((H, 40), lambda i: (0, 0)),
            pl.BlockSpec((1, 40), lambda i: (0, 0)),
            pl.BlockSpec((40, 3), lambda i: (0, 0)),
            pl.BlockSpec((1, 3), lambda i: (0, 0)),
        ],
        out_specs=pl.BlockSpec((1, 3), lambda i: (0, 0)),
        out_shape=jax.ShapeDtypeStruct((1, 3), jnp.float32),
        scratch_shapes=[pltpu.VMEM((1, H), jnp.float32)],
    )(acc0, acc1, g0, dinv, t0, t1, b1, W2, b2, W3, b3)


def kernel(x, edge_index, W1, b1, W2, b2, W3, b3):
    src = edge_index[0]
    dst = edge_index[1]
    pad = (N + (jnp.arange(EP - E, dtype=jnp.int32) % 128)).astype(jnp.int32)
    srcp = jnp.concatenate([src, pad]).reshape(NT, NCHUNK, CHUNK)
    dstp = jnp.concatenate([dst, pad]).reshape(NT, NCHUNK, CHUNK)
    xp = jnp.pad(x, ((0, NP - N), (0, 0)))
    z1 = jnp.zeros((STRIPE,), jnp.float32)
    zH = jnp.zeros((STRIPE, H), jnp.float32)

    deg2 = _sc_deg(dstp, z1)
    deg0 = deg2[:NP].reshape(NP, 1)
    deg1 = deg2[NP:].reshape(NP, 1)
    g0, dinv = _tc1(xp, W1, deg0, deg1)
    acc2, t2 = _sc_edge(g0, dinv.reshape(NP), srcp, dstp, zH, z1)
    out = _tc2(acc2[:NP], acc2[NP:], g0, dinv,
               t2[:NP].reshape(NP, 1), t2[NP:].reshape(NP, 1),
               b1.reshape(1, H), W2, b2.reshape(1, 40), W3, b3.reshape(1, 3))
    return out


# speedup vs baseline: 25.2820x; 25.2820x over previous
import functools

import jax
import jax.numpy as jnp
from jax import lax
from jax.experimental import pallas as pl
from jax.experimental.pallas import tpu as pltpu, tpu_sc as plsc

N = 10000
D = 256
H = 16
E = 160000
NP = 10240
NT = 32
CHUNK = 128
NCHUNK = 40
EPT = CHUNK * NCHUNK
EP = NT * EPT
STRIPE = NP // 16

_mesh = plsc.VectorSubcoreMesh(core_axis_name="c", subcore_axis_name="s")


@functools.partial(
    pl.kernel,
    out_type=jax.ShapeDtypeStruct((2 * NP,), jnp.float32),
    mesh=_mesh,
    scratch_types=[
        pltpu.VMEM((NCHUNK, CHUNK), jnp.int32),
        pltpu.VMEM((CHUNK,), jnp.float32),
        pltpu.VMEM_SHARED((NP,), jnp.float32),
    ],
)
def _sc_deg(dst_hbm, z1_hbm, deg_out, idx_v, ones_v, deg_s):
    cid = lax.axis_index("c")
    sid = lax.axis_index("s")
    wid = sid * 2 + cid
    pltpu.sync_copy(dst_hbm.at[wid], idx_v)
    for k in range(CHUNK // 16):
        ones_v[pl.ds(k * 16, 16)] = jnp.ones((16,), jnp.float32)
    pltpu.sync_copy(z1_hbm, deg_s.at[pl.ds(sid * STRIPE, STRIPE)])
    plsc.subcore_barrier()

    def body(j, carry):
        pltpu.sync_copy(ones_v, deg_s.at[idx_v.at[j]], add=True)
        return carry

    lax.fori_loop(0, NCHUNK, body, 0)
    plsc.subcore_barrier()
    pltpu.sync_copy(deg_s.at[pl.ds(sid * STRIPE, STRIPE)],
                    deg_out.at[pl.ds(cid * NP + sid * STRIPE, STRIPE)])


_ROWS = 1024


def _tc1_body(x_ref, w1_ref, d0_ref, d1_ref, g0_ref, dinv_ref):
    h0 = jnp.dot(x_ref[...], w1_ref[...], preferred_element_type=jnp.float32)
    deg = d0_ref[...] + d1_ref[...] + 1.0
    dinv = lax.rsqrt(deg)
    dinv_ref[...] = dinv
    g0_ref[...] = h0 * dinv


def _tc1(xp, W1, deg0, deg1):
    return pl.pallas_call(
        _tc1_body,
        grid=(NP // _ROWS,),
        in_specs=[
            pl.BlockSpec((_ROWS, D), lambda i: (i, 0)),
            pl.BlockSpec((D, H), lambda i: (0, 0)),
            pl.BlockSpec((_ROWS, 1), lambda i: (i, 0)),
            pl.BlockSpec((_ROWS, 1), lambda i: (i, 0)),
        ],
        out_specs=[
            pl.BlockSpec((_ROWS, H), lambda i: (i, 0)),
            pl.BlockSpec((_ROWS, 1), lambda i: (i, 0)),
        ],
        out_shape=[
            jax.ShapeDtypeStruct((NP, H), jnp.float32),
            jax.ShapeDtypeStruct((NP, 1), jnp.float32),
        ],
    )(xp, W1, deg0, deg1)


@functools.partial(
    pl.kernel,
    out_type=[
        jax.ShapeDtypeStruct((2 * NP, H), jnp.float32),
        jax.ShapeDtypeStruct((2 * NP,), jnp.float32),
    ],
    mesh=_mesh,
    scratch_types=[
        pltpu.VMEM((NCHUNK, CHUNK), jnp.int32),
        pltpu.VMEM((NCHUNK, CHUNK), jnp.int32),
        pltpu.VMEM((CHUNK, H), jnp.float32),
        pltpu.VMEM((CHUNK,), jnp.float32),
        pltpu.VMEM_SHARED((NP, H), jnp.float32),
        pltpu.VMEM_SHARED((NP,), jnp.float32),
        pltpu.SemaphoreType.DMA,
    ],
    compiler_params=pltpu.CompilerParams(use_tc_tiling_on_sc=False),
)
def _sc_edge(g0_hbm, dinv_hbm, src_hbm, dst_hbm, zH_hbm, z1_hbm,
             acc_out, t_out,
             src_v, dst_v, rows_v, tvals_v, acc_s, t_s, sem):
    cid = lax.axis_index("c")
    sid = lax.axis_index("s")
    wid = sid * 2 + cid
    pltpu.sync_copy(src_hbm.at[wid], src_v)
    pltpu.sync_copy(dst_hbm.at[wid], dst_v)
    pltpu.sync_copy(zH_hbm, acc_s.at[pl.ds(sid * STRIPE, STRIPE), :])
    pltpu.sync_copy(z1_hbm, t_s.at[pl.ds(sid * STRIPE, STRIPE)])
    plsc.subcore_barrier()

    def body(j, carry):
        pltpu.async_copy(g0_hbm.at[src_v.at[j]], rows_v, sem).wait()
        pltpu.sync_copy(rows_v, acc_s.at[dst_v.at[j]], add=True)
        pltpu.async_copy(dinv_hbm.at[dst_v.at[j]], tvals_v, sem).wait()
        pltpu.sync_copy(tvals_v, t_s.at[src_v.at[j]], add=True)
        return carry

    lax.fori_loop(0, NCHUNK, body, 0)
    plsc.subcore_barrier()
    pltpu.sync_copy(acc_s.at[pl.ds(sid * STRIPE, STRIPE), :],
                    acc_out.at[pl.ds(cid * NP + sid * STRIPE, STRIPE), :])
    pltpu.sync_copy(t_s.at[pl.ds(sid * STRIPE, STRIPE)],
                    t_out.at[pl.ds(cid * NP + sid * STRIPE, STRIPE)])


def _tc2_body(a0_ref, a1_ref, g0_ref, dinv_ref, t0_ref, t1_ref,
              b1_ref, w2_ref, b2_ref, w3_ref, b3_ref, out_ref, vacc):
    i = pl.program_id(0)
    dinv = dinv_ref[...]
    acc = a0_ref[...] + a1_ref[...] + g0_ref[...]
    h1 = jnp.maximum(dinv * acc + b1_ref[...], 0.0)
    w = dinv * (t0_ref[...] + t1_ref[...] + dinv)
    ridx = lax.broadcasted_iota(jnp.int32, (_ROWS, 1), 0) + i * _ROWS
    w = jnp.where(ridx < N, w, 0.0)
    part = jnp.sum(h1 * w, axis=0, keepdims=True)

    @pl.when(i == 0)
    def _():
        vacc[...] = part

    @pl.when(i > 0)
    def _():
        vacc[...] = vacc[...] + part

    @pl.when(i == NP // _ROWS - 1)
    def _():
        v = vacc[...] * (1.0 / N)
        pooled = jnp.dot(v, w2_ref[...], preferred_element_type=jnp.float32) + b2_ref[...]
        out_ref[...] = jnp.dot(pooled, w3_ref[...], preferred_element_type=jnp.float32) + b3_ref[...]


def _tc2(acc0, acc1, g0, dinv, t0, t1, b1, W2, b2, W3, b3):
    return pl.pallas_call(
        _tc2_body,
        grid=(NP // _ROWS,),
        in_specs=[
            pl.BlockSpec((_ROWS, H), lambda i: (i, 0)),
            pl.BlockSpec((_ROWS, H), lambda i: (i, 0)),
            pl.BlockSpec((_ROWS, H), lambda i: (i, 0)),
            pl.BlockSpec((_ROWS, 1), lambda i: (i, 0)),
            pl.BlockSpec((_ROWS, 1), lambda i: (i, 0)),
            pl.BlockSpec((_ROWS, 1), lambda i: (i, 0)),
            pl.BlockSpec((1, H), lambda i: (0, 0)),
            pl.BlockSpec((H, 40), lambda i: (0, 0)),
            pl.BlockSpec((1, 40), lambda i: (0, 0)),
            pl.BlockSpec((40, 3), lambda i: (0, 0)),
            pl.BlockSpec((1, 3), lambda i: (0, 0)),
        ],
        out_specs=pl.BlockSpec((1, 3), lambda i: (0, 0)),
        out_shape=jax.ShapeDtypeStruct((1, 3), jnp.float32),
        scratch_shapes=[pltpu.VMEM((1, H), jnp.float32)],
    )(acc0, acc1, g0, dinv, t0, t1, b1, W2, b2, W3, b3)


def kernel(x, edge_index, W1, b1, W2, b2, W3, b3):
    src = edge_index[0]
    dst = edge_index[1]
    pad = (N + (jnp.arange(EP - E, dtype=jnp.int32) % 128)).astype(jnp.int32)
    srcp = jnp.concatenate([src, pad]).reshape(NT, NCHUNK, CHUNK)
    dstp = jnp.concatenate([dst, pad]).reshape(NT, NCHUNK, CHUNK)
    xp = jnp.pad(x, ((0, NP - N), (0, 0)))
    z1 = jnp.zeros((STRIPE,), jnp.float32)
    zH = jnp.zeros((STRIPE, H), jnp.float32)

    deg2 = _sc_deg(dstp, z1)
    deg0 = deg2[:NP].reshape(NP, 1)
    deg1 = deg2[NP:].reshape(NP, 1)
    g0, dinv = _tc1(xp, W1, deg0, deg1)
    acc2, t2 = _sc_edge(g0, dinv.reshape(NP), srcp, dstp, zH, z1)
    out = _tc2(acc2[:NP], acc2[NP:], g0, dinv,
               t2[:NP].reshape(NP, 1), t2[NP:].reshape(NP, 1),
               b1.reshape(1, H), W2, b2.reshape(1, 40), W3, b3.reshape(1, 3))
    return out


# speedup vs baseline: 32.6045x; 1.2896x over previous
import functools

import jax
import jax.numpy as jnp
from jax import lax
from jax.experimental import pallas as pl
from jax.experimental.pallas import tpu as pltpu, tpu_sc as plsc

N = 10000
D = 256
H = 16
E = 160000
NP = 10240
NT = 32
CHUNK = 128
NCHUNK = 40
EPT = CHUNK * NCHUNK
EP = NT * EPT
STRIPE = NP // 16

_mesh = plsc.VectorSubcoreMesh(core_axis_name="c", subcore_axis_name="s")


@functools.partial(
    pl.kernel,
    out_type=jax.ShapeDtypeStruct((2 * NP,), jnp.float32),
    mesh=_mesh,
    scratch_types=[
        pltpu.VMEM((NCHUNK, CHUNK), jnp.int32),
        pltpu.VMEM((CHUNK,), jnp.float32),
        pltpu.VMEM_SHARED((NP,), jnp.float32),
        pltpu.SemaphoreType.DMA,
    ],
)
def _sc_deg(dst_hbm, z1_hbm, deg_out, idx_v, ones_v, deg_s, sem):
    cid = lax.axis_index("c")
    sid = lax.axis_index("s")
    wid = sid * 2 + cid
    pltpu.sync_copy(dst_hbm.at[wid], idx_v)
    for k in range(CHUNK // 16):
        ones_v[pl.ds(k * 16, 16)] = jnp.ones((16,), jnp.float32)
    pltpu.sync_copy(z1_hbm, deg_s.at[pl.ds(sid * STRIPE, STRIPE)])
    plsc.subcore_barrier()

    descs = [pltpu.async_copy(ones_v, deg_s.at[idx_v.at[j]], sem, add=True)
             for j in range(NCHUNK)]
    for d in descs:
        d.wait()
    plsc.subcore_barrier()
    pltpu.sync_copy(deg_s.at[pl.ds(sid * STRIPE, STRIPE)],
                    deg_out.at[pl.ds(cid * NP + sid * STRIPE, STRIPE)])


_ROWS = 1024


def _tc1_body(x_ref, w1_ref, d0_ref, d1_ref, g0_ref, dinv_ref):
    h0 = jnp.dot(x_ref[...], w1_ref[...], preferred_element_type=jnp.float32)
    deg = d0_ref[...] + d1_ref[...] + 1.0
    dinv = lax.rsqrt(deg)
    dinv_ref[...] = dinv
    g0_ref[...] = h0 * dinv


def _tc1(xp, W1, deg0, deg1):
    return pl.pallas_call(
        _tc1_body,
        grid=(NP // _ROWS,),
        in_specs=[
            pl.BlockSpec((_ROWS, D), lambda i: (i, 0)),
            pl.BlockSpec((D, H), lambda i: (0, 0)),
            pl.BlockSpec((_ROWS, 1), lambda i: (i, 0)),
            pl.BlockSpec((_ROWS, 1), lambda i: (i, 0)),
        ],
        out_specs=[
            pl.BlockSpec((_ROWS, H), lambda i: (i, 0)),
            pl.BlockSpec((_ROWS, 1), lambda i: (i, 0)),
        ],
        out_shape=[
            jax.ShapeDtypeStruct((NP, H), jnp.float32),
            jax.ShapeDtypeStruct((NP, 1), jnp.float32),
        ],
    )(xp, W1, deg0, deg1)


@functools.partial(
    pl.kernel,
    out_type=[
        jax.ShapeDtypeStruct((2 * NP, H), jnp.float32),
        jax.ShapeDtypeStruct((2 * NP,), jnp.float32),
    ],
    mesh=_mesh,
    scratch_types=[
        pltpu.VMEM((NCHUNK, CHUNK), jnp.int32),
        pltpu.VMEM((NCHUNK, CHUNK), jnp.int32),
        pltpu.VMEM((NCHUNK, CHUNK, H), jnp.float32),
        pltpu.VMEM((NCHUNK, CHUNK), jnp.float32),
        pltpu.VMEM_SHARED((NP, H), jnp.float32),
        pltpu.VMEM_SHARED((NP,), jnp.float32),
        pltpu.SemaphoreType.DMA,
        pltpu.SemaphoreType.DMA,
        pltpu.SemaphoreType.DMA,
        pltpu.SemaphoreType.DMA,
    ],
    compiler_params=pltpu.CompilerParams(use_tc_tiling_on_sc=False),
)
def _sc_edge(g0_hbm, dinv_hbm, src_hbm, dst_hbm, zH_hbm, z1_hbm,
             acc_out, t_out,
             src_v, dst_v, rows_v, tvals_v, acc_s, t_s,
             gsem, tsem, s1sem, s2sem):
    cid = lax.axis_index("c")
    sid = lax.axis_index("s")
    wid = sid * 2 + cid
    pltpu.sync_copy(src_hbm.at[wid], src_v)
    pltpu.sync_copy(dst_hbm.at[wid], dst_v)
    pltpu.sync_copy(zH_hbm, acc_s.at[pl.ds(sid * STRIPE, STRIPE), :])
    pltpu.sync_copy(z1_hbm, t_s.at[pl.ds(sid * STRIPE, STRIPE)])
    plsc.subcore_barrier()

    gds = []
    tds = []
    for j in range(NCHUNK):
        gds.append(pltpu.async_copy(g0_hbm.at[src_v.at[j]], rows_v.at[j], gsem))
        tds.append(pltpu.async_copy(dinv_hbm.at[dst_v.at[j]], tvals_v.at[j], tsem))
    sds = []
    for j in range(NCHUNK):
        gds[j].wait()
        sds.append(pltpu.async_copy(rows_v.at[j], acc_s.at[dst_v.at[j]], s1sem, add=True))
        tds[j].wait()
        sds.append(pltpu.async_copy(tvals_v.at[j], t_s.at[src_v.at[j]], s2sem, add=True))
    for d in sds:
        d.wait()
    plsc.subcore_barrier()
    pltpu.sync_copy(acc_s.at[pl.ds(sid * STRIPE, STRIPE), :],
                    acc_out.at[pl.ds(cid * NP + sid * STRIPE, STRIPE), :])
    pltpu.sync_copy(t_s.at[pl.ds(sid * STRIPE, STRIPE)],
                    t_out.at[pl.ds(cid * NP + sid * STRIPE, STRIPE)])


def _tc2_body(a0_ref, a1_ref, g0_ref, dinv_ref, t0_ref, t1_ref,
              b1_ref, w2_ref, b2_ref, w3_ref, b3_ref, out_ref, vacc):
    i = pl.program_id(0)
    dinv = dinv_ref[...]
    acc = a0_ref[...] + a1_ref[...] + g0_ref[...]
    h1 = jnp.maximum(dinv * acc + b1_ref[...], 0.0)
    w = dinv * (t0_ref[...] + t1_ref[...] + dinv)
    ridx = lax.broadcasted_iota(jnp.int32, (_ROWS, 1), 0) + i * _ROWS
    w = jnp.where(ridx < N, w, 0.0)
    part = jnp.sum(h1 * w, axis=0, keepdims=True)

    @pl.when(i == 0)
    def _():
        vacc[...] = part

    @pl.when(i > 0)
    def _():
        vacc[...] = vacc[...] + part

    @pl.when(i == NP // _ROWS - 1)
    def _():
        v = vacc[...] * (1.0 / N)
        pooled = jnp.dot(v, w2_ref[...], preferred_element_type=jnp.float32) + b2_ref[...]
        out_ref[...] = jnp.dot(pooled, w3_ref[...], preferred_element_type=jnp.float32) + b3_ref[...]


def _tc2(acc0, acc1, g0, dinv, t0, t1, b1, W2, b2, W3, b3):
    return pl.pallas_call(
        _tc2_body,
        grid=(NP // _ROWS,),
        in_specs=[
            pl.BlockSpec((_ROWS, H), lambda i: (i, 0)),
            pl.BlockSpec((_ROWS, H), lambda i: (i, 0)),
            pl.BlockSpec((_ROWS, H), lambda i: (i, 0)),
            pl.BlockSpec((_ROWS, 1), lambda i: (i, 0)),
            pl.BlockSpec((_ROWS, 1), lambda i: (i, 0)),
            pl.BlockSpec((_ROWS, 1), lambda i: (i, 0)),
            pl.BlockSpec((1, H), lambda i: (0, 0)),
            pl.BlockSpec((H, 40), lambda i: (0, 0)),
            pl.BlockSpec((1, 40), lambda i: (0, 0)),
            pl.BlockSpec((40, 3), lambda i: (0, 0)),
            pl.BlockSpec((1, 3), lambda i: (0, 0)),
        ],
        out_specs=pl.BlockSpec((1, 3), lambda i: (0, 0)),
        out_shape=jax.ShapeDtypeStruct((1, 3), jnp.float32),
        scratch_shapes=[pltpu.VMEM((1, H), jnp.float32)],
    )(acc0, acc1, g0, dinv, t0, t1, b1, W2, b2, W3, b3)


def kernel(x, edge_index, W1, b1, W2, b2, W3, b3):
    src = edge_index[0]
    dst = edge_index[1]
    pad = (N + (jnp.arange(EP - E, dtype=jnp.int32) % 128)).astype(jnp.int32)
    srcp = jnp.concatenate([src, pad]).reshape(NT, NCHUNK, CHUNK)
    dstp = jnp.concatenate([dst, pad]).reshape(NT, NCHUNK, CHUNK)
    xp = jnp.pad(x, ((0, NP - N), (0, 0)))
    z1 = jnp.zeros((STRIPE,), jnp.float32)
    zH = jnp.zeros((STRIPE, H), jnp.float32)

    deg2 = _sc_deg(dstp, z1)
    deg0 = deg2[:NP].reshape(NP, 1)
    deg1 = deg2[NP:].reshape(NP, 1)
    g0, dinv = _tc1(xp, W1, deg0, deg1)
    acc2, t2 = _sc_edge(g0, dinv.reshape(NP), srcp, dstp, zH, z1)
    out = _tc2(acc2[:NP], acc2[NP:], g0, dinv,
               t2[:NP].reshape(NP, 1), t2[NP:].reshape(NP, 1),
               b1.reshape(1, H), W2, b2.reshape(1, 40), W3, b3.reshape(1, 3))
    return out


# speedup vs baseline: 36.2626x; 1.1122x over previous
import functools

import jax
import jax.numpy as jnp
from jax import lax
from jax.experimental import pallas as pl
from jax.experimental.pallas import tpu as pltpu, tpu_sc as plsc

N = 10000
D = 256
H = 16
E = 160000
NP = 10240
NT = 32
CHUNK = 128
NCHUNK = 40
EP = NT * NCHUNK * CHUNK
STRIPE = NP // 16

_mesh = plsc.VectorSubcoreMesh(core_axis_name="c", subcore_axis_name="s")


@functools.partial(
    pl.kernel,
    out_type=[
        jax.ShapeDtypeStruct((NP,), jnp.float32),
        jax.ShapeDtypeStruct((NP,), jnp.float32),
    ],
    mesh=_mesh,
    scratch_types=[
        pltpu.VMEM((NCHUNK, CHUNK), jnp.int32),
        pltpu.VMEM((CHUNK,), jnp.float32),
        pltpu.VMEM_SHARED((NP,), jnp.float32),
        pltpu.SemaphoreType.DMA,
    ],
)
def _sc_deg(dst_hbm, z1_hbm, deg0_out, deg1_out, idx_v, ones_v, deg_s, sem):
    cid = lax.axis_index("c")
    sid = lax.axis_index("s")
    wid = sid * 2 + cid
    pltpu.sync_copy(dst_hbm.at[wid], idx_v)
    for k in range(CHUNK // 16):
        ones_v[pl.ds(k * 16, 16)] = jnp.ones((16,), jnp.float32)
    pltpu.sync_copy(z1_hbm, deg_s.at[pl.ds(sid * STRIPE, STRIPE)])
    plsc.subcore_barrier()

    descs = [pltpu.async_copy(ones_v, deg_s.at[idx_v.at[j]], sem, add=True)
             for j in range(NCHUNK)]
    for d in descs:
        d.wait()
    plsc.subcore_barrier()

    @pl.when(cid == 0)
    def _():
        pltpu.sync_copy(deg_s.at[pl.ds(sid * STRIPE, STRIPE)],
                        deg0_out.at[pl.ds(sid * STRIPE, STRIPE)])

    @pl.when(cid == 1)
    def _():
        pltpu.sync_copy(deg_s.at[pl.ds(sid * STRIPE, STRIPE)],
                        deg1_out.at[pl.ds(sid * STRIPE, STRIPE)])


_R1 = 1024


def _tc1a_body(x_ref, w1_ref, h0_ref):
    h0_ref[...] = jnp.dot(x_ref[...], w1_ref[...],
                          preferred_element_type=jnp.float32)


def _tc1a(x, W1):
    return pl.pallas_call(
        _tc1a_body,
        grid=(NP // _R1,),
        in_specs=[
            pl.BlockSpec((_R1, D), lambda i: (i, 0)),
            pl.BlockSpec((D, H), lambda i: (0, 0)),
        ],
        out_specs=pl.BlockSpec((_R1, H), lambda i: (i, 0)),
        out_shape=jax.ShapeDtypeStruct((NP, H), jnp.float32),
    )(x, W1)


def _tc1b_body(h0_ref, d0_ref, d1_ref, g0_ref, dinv_ref):
    deg = d0_ref[...] + d1_ref[...] + 1.0
    dinv = lax.rsqrt(deg)
    dinv_ref[...] = dinv
    g0_ref[...] = h0_ref[...] * dinv


def _tc1b(h0, deg0, deg1):
    return pl.pallas_call(
        _tc1b_body,
        grid=(NP // _R1,),
        in_specs=[
            pl.BlockSpec((_R1, H), lambda i: (i, 0)),
            pl.BlockSpec((_R1, 1), lambda i: (i, 0)),
            pl.BlockSpec((_R1, 1), lambda i: (i, 0)),
        ],
        out_specs=[
            pl.BlockSpec((_R1, H), lambda i: (i, 0)),
            pl.BlockSpec((_R1, 1), lambda i: (i, 0)),
        ],
        out_shape=[
            jax.ShapeDtypeStruct((NP, H), jnp.float32),
            jax.ShapeDtypeStruct((NP, 1), jnp.float32),
        ],
    )(h0, deg0, deg1)


_NB = 12


@functools.partial(
    pl.kernel,
    out_type=[
        jax.ShapeDtypeStruct((NP, H), jnp.float32),
        jax.ShapeDtypeStruct((NP, H), jnp.float32),
        jax.ShapeDtypeStruct((NP,), jnp.float32),
        jax.ShapeDtypeStruct((NP,), jnp.float32),
    ],
    mesh=_mesh,
    scratch_types=[
        pltpu.VMEM((NCHUNK, CHUNK), jnp.int32),
        pltpu.VMEM((NCHUNK, CHUNK), jnp.int32),
        pltpu.VMEM((_NB, CHUNK, H), jnp.float32),
        pltpu.VMEM((NCHUNK, CHUNK), jnp.float32),
        pltpu.VMEM_SHARED((NP, H), jnp.float32),
        pltpu.VMEM_SHARED((NP,), jnp.float32),
        pltpu.SemaphoreType.DMA,
        pltpu.SemaphoreType.DMA,
        pltpu.SemaphoreType.DMA,
        pltpu.SemaphoreType.DMA,
    ],
    compiler_params=pltpu.CompilerParams(use_tc_tiling_on_sc=False),
)
def _sc_edge(g0_hbm, dinv_hbm, src_hbm, dst_hbm, zH_hbm, z1_hbm,
             acc0_out, acc1_out, t0_out, t1_out,
             src_v, dst_v, rows_v, tvals_v, acc_s, t_s,
             gsem, tsem, s1sem, s2sem):
    cid = lax.axis_index("c")
    sid = lax.axis_index("s")
    wid = sid * 2 + cid
    pltpu.sync_copy(src_hbm.at[wid], src_v)
    pltpu.sync_copy(dst_hbm.at[wid], dst_v)
    pltpu.sync_copy(zH_hbm, acc_s.at[pl.ds(sid * STRIPE, STRIPE), :])
    pltpu.sync_copy(z1_hbm, t_s.at[pl.ds(sid * STRIPE, STRIPE)])
    plsc.subcore_barrier()

    gds = {}
    for j in range(_NB):
        gds[j] = pltpu.async_copy(g0_hbm.at[src_v.at[j]], rows_v.at[j], gsem)
    tds = [pltpu.async_copy(dinv_hbm.at[dst_v.at[j]], tvals_v.at[j], tsem)
           for j in range(NCHUNK)]
    sds = []
    s2ds = []
    for j in range(NCHUNK):
        gds[j].wait()
        d = pltpu.async_copy(rows_v.at[j % _NB], acc_s.at[dst_v.at[j]], s1sem, add=True)
        if j + _NB < NCHUNK:
            d.wait()
            gds[j + _NB] = pltpu.async_copy(
                g0_hbm.at[src_v.at[j + _NB]], rows_v.at[(j + _NB) % _NB], gsem)
        else:
            sds.append(d)
        tds[j].wait()
        s2ds.append(pltpu.async_copy(tvals_v.at[j], t_s.at[src_v.at[j]], s2sem, add=True))
    for d in sds:
        d.wait()
    for d in s2ds:
        d.wait()
    plsc.subcore_barrier()

    @pl.when(cid == 0)
    def _():
        pltpu.sync_copy(acc_s.at[pl.ds(sid * STRIPE, STRIPE), :],
                        acc0_out.at[pl.ds(sid * STRIPE, STRIPE), :])
        pltpu.sync_copy(t_s.at[pl.ds(sid * STRIPE, STRIPE)],
                        t0_out.at[pl.ds(sid * STRIPE, STRIPE)])

    @pl.when(cid == 1)
    def _():
        pltpu.sync_copy(acc_s.at[pl.ds(sid * STRIPE, STRIPE), :],
                        acc1_out.at[pl.ds(sid * STRIPE, STRIPE), :])
        pltpu.sync_copy(t_s.at[pl.ds(sid * STRIPE, STRIPE)],
                        t1_out.at[pl.ds(sid * STRIPE, STRIPE)])


_R2 = 5000


def _tc2_body(a0_ref, a1_ref, g0_ref, dinv_ref, t0_ref, t1_ref,
              b1_ref, w2_ref, b2_ref, w3_ref, b3_ref, out_ref, vacc):
    i = pl.program_id(0)
    dinv = dinv_ref[...]
    acc = a0_ref[...] + a1_ref[...] + g0_ref[...]
    h1 = jnp.maximum(dinv * acc + b1_ref[...], 0.0)
    w = dinv * (t0_ref[...] + t1_ref[...] + dinv)
    part = jnp.sum(h1 * w, axis=0, keepdims=True)

    @pl.when(i == 0)
    def _():
        vacc[...] = part

    @pl.when(i == N // _R2 - 1)
    def _():
        v = (vacc[...] + part) * (1.0 / N)
        pooled = jnp.dot(v, w2_ref[...], preferred_element_type=jnp.float32) + b2_ref[...]
        out_ref[...] = jnp.dot(pooled, w3_ref[...], preferred_element_type=jnp.float32) + b3_ref[...]


def _tc2(acc0, acc1, g0, dinv, t0, t1, b1, W2, b2, W3, b3):
    return pl.pallas_call(
        _tc2_body,
        grid=(N // _R2,),
        in_specs=[
            pl.BlockSpec((_R2, H), lambda i: (i, 0)),
            pl.BlockSpec((_R2, H), lambda i: (i, 0)),
            pl.BlockSpec((_R2, H), lambda i: (i, 0)),
            pl.BlockSpec((_R2, 1), lambda i: (i, 0)),
            pl.BlockSpec((_R2, 1), lambda i: (i, 0)),
            pl.BlockSpec((_R2, 1), lambda i: (i, 0)),
            pl.BlockSpec((1, H), lambda i: (0, 0)),
            pl.BlockSpec((H, 40), lambda i: (0, 0)),
            pl.BlockSpec((1, 40), lambda i: (0, 0)),
            pl.BlockSpec((40, 3), lambda i: (0, 0)),
            pl.BlockSpec((1, 3), lambda i: (0, 0)),
        ],
        out_specs=pl.BlockSpec((1, 3), lambda i: (0, 0)),
        out_shape=jax.ShapeDtypeStruct((1, 3), jnp.float32),
        scratch_shapes=[pltpu.VMEM((1, H), jnp.float32)],
    )(acc0, acc1, g0, dinv, t0, t1, b1, W2, b2, W3, b3)


def kernel(x, edge_index, W1, b1, W2, b2, W3, b3):
    pad = (N + (jnp.arange(EP - E, dtype=jnp.int32) % 128)).astype(jnp.int32)
    srcp = jnp.concatenate([edge_index[0], pad]).reshape(NT, NCHUNK, CHUNK)
    dstp = jnp.concatenate([edge_index[1], pad]).reshape(NT, NCHUNK, CHUNK)
    z1 = jnp.zeros((STRIPE,), jnp.float32)
    zH = jnp.zeros((STRIPE, H), jnp.float32)

    deg0, deg1 = _sc_deg(dstp, z1)
    h0 = _tc1a(x, W1)
    g0, dinv = _tc1b(h0, deg0.reshape(NP, 1), deg1.reshape(NP, 1))
    acc0, acc1, t0, t1 = _sc_edge(g0, dinv.reshape(NP), srcp, dstp, zH, z1)
    return _tc2(acc0, acc1, g0, dinv, t0.reshape(NP, 1), t1.reshape(NP, 1),
                b1.reshape(1, H), W2, b2.reshape(1, 40), W3, b3.reshape(1, 3))


# speedup vs baseline: 42.4912x; 1.1718x over previous
import functools

import jax
import jax.numpy as jnp
from jax import lax
from jax.experimental import pallas as pl
from jax.experimental.pallas import tpu as pltpu, tpu_sc as plsc

N = 10000
D = 256
H = 16
E = 160000
NP = 10240
NT = 32
CHUNK = 128
NCHUNK = 40
EP = NT * NCHUNK * CHUNK
STRIPE = NP // 16

_mesh = plsc.VectorSubcoreMesh(core_axis_name="c", subcore_axis_name="s")


@functools.partial(
    pl.kernel,
    out_type=[
        jax.ShapeDtypeStruct((NP,), jnp.float32),
        jax.ShapeDtypeStruct((NP,), jnp.float32),
    ],
    mesh=_mesh,
    scratch_types=[
        pltpu.VMEM((NCHUNK, CHUNK), jnp.int32),
        pltpu.VMEM((CHUNK,), jnp.float32),
        pltpu.VMEM_SHARED((NP,), jnp.float32),
        pltpu.SemaphoreType.DMA,
    ],
)
def _sc_deg(dst_hbm, z1_hbm, deg0_out, deg1_out, idx_v, ones_v, deg_s, sem):
    cid = lax.axis_index("c")
    sid = lax.axis_index("s")
    wid = sid * 2 + cid
    pltpu.sync_copy(dst_hbm.at[wid], idx_v)
    for k in range(CHUNK // 16):
        ones_v[pl.ds(k * 16, 16)] = jnp.ones((16,), jnp.float32)
    pltpu.sync_copy(z1_hbm, deg_s.at[pl.ds(sid * STRIPE, STRIPE)])
    plsc.subcore_barrier()

    descs = [pltpu.async_copy(ones_v, deg_s.at[idx_v.at[j]], sem, add=True)
             for j in range(NCHUNK)]
    for d in descs:
        d.wait()
    plsc.subcore_barrier()

    @pl.when(cid == 0)
    def _():
        pltpu.sync_copy(deg_s.at[pl.ds(sid * STRIPE, STRIPE)],
                        deg0_out.at[pl.ds(sid * STRIPE, STRIPE)])

    @pl.when(cid == 1)
    def _():
        pltpu.sync_copy(deg_s.at[pl.ds(sid * STRIPE, STRIPE)],
                        deg1_out.at[pl.ds(sid * STRIPE, STRIPE)])


_R1 = 1024


def _tc1a_body(x_ref, w1_ref, h0_ref):
    h0_ref[...] = jnp.dot(x_ref[...], w1_ref[...],
                          preferred_element_type=jnp.float32)


def _tc1a(x, W1):
    return pl.pallas_call(
        _tc1a_body,
        grid=(NP // _R1,),
        in_specs=[
            pl.BlockSpec((_R1, D), lambda i: (i, 0)),
            pl.BlockSpec((D, H), lambda i: (0, 0)),
        ],
        out_specs=pl.BlockSpec((_R1, H), lambda i: (i, 0)),
        out_shape=jax.ShapeDtypeStruct((NP, H), jnp.float32),
    )(x, W1)


def _tc1b_body(h0_ref, d0_ref, d1_ref, g0_ref, dinv1_ref, dinv2_ref):
    deg = d0_ref[...] + d1_ref[...] + 1.0
    dinv = lax.rsqrt(deg)
    dinv1_ref[...] = dinv
    dinv2 = dinv.reshape(_R1, 1)
    dinv2_ref[...] = dinv2
    g0_ref[...] = h0_ref[...] * dinv2


def _tc1b(h0, deg0, deg1):
    return pl.pallas_call(
        _tc1b_body,
        grid=(NP // _R1,),
        in_specs=[
            pl.BlockSpec((_R1, H), lambda i: (i, 0)),
            pl.BlockSpec((_R1,), lambda i: (i,)),
            pl.BlockSpec((_R1,), lambda i: (i,)),
        ],
        out_specs=[
            pl.BlockSpec((_R1, H), lambda i: (i, 0)),
            pl.BlockSpec((_R1,), lambda i: (i,)),
            pl.BlockSpec((_R1, 1), lambda i: (i, 0)),
        ],
        out_shape=[
            jax.ShapeDtypeStruct((NP, H), jnp.float32),
            jax.ShapeDtypeStruct((NP,), jnp.float32),
            jax.ShapeDtypeStruct((NP, 1), jnp.float32),
        ],
    )(h0, deg0, deg1)


_NB = 24


@functools.partial(
    pl.kernel,
    out_type=[
        jax.ShapeDtypeStruct((NP, H), jnp.float32),
        jax.ShapeDtypeStruct((NP, H), jnp.float32),
        jax.ShapeDtypeStruct((NP,), jnp.float32),
        jax.ShapeDtypeStruct((NP,), jnp.float32),
    ],
    mesh=_mesh,
    scratch_types=[
        pltpu.VMEM((NCHUNK, CHUNK), jnp.int32),
        pltpu.VMEM((NCHUNK, CHUNK), jnp.int32),
        pltpu.VMEM((_NB, CHUNK, H), jnp.float32),
        pltpu.VMEM((NCHUNK, CHUNK), jnp.float32),
        pltpu.VMEM_SHARED((NP, H), jnp.float32),
        pltpu.VMEM_SHARED((NP,), jnp.float32),
        pltpu.SemaphoreType.DMA,
        pltpu.SemaphoreType.DMA,
        pltpu.SemaphoreType.DMA,
        pltpu.SemaphoreType.DMA,
    ],
    compiler_params=pltpu.CompilerParams(use_tc_tiling_on_sc=False),
)
def _sc_edge(g0_hbm, dinv_hbm, src_hbm, dst_hbm, zH_hbm, z1_hbm,
             acc0_out, acc1_out, t0_out, t1_out,
             src_v, dst_v, rows_v, tvals_v, acc_s, t_s,
             gsem, tsem, s1sem, s2sem):
    cid = lax.axis_index("c")
    sid = lax.axis_index("s")
    wid = sid * 2 + cid
    pltpu.sync_copy(src_hbm.at[wid], src_v)
    pltpu.sync_copy(dst_hbm.at[wid], dst_v)
    pltpu.sync_copy(zH_hbm, acc_s.at[pl.ds(sid * STRIPE, STRIPE), :])
    pltpu.sync_copy(z1_hbm, t_s.at[pl.ds(sid * STRIPE, STRIPE)])
    plsc.subcore_barrier()

    gds = {}
    for j in range(_NB):
        gds[j] = pltpu.async_copy(g0_hbm.at[src_v.at[j]], rows_v.at[j], gsem)
    tds = [pltpu.async_copy(dinv_hbm.at[dst_v.at[j]], tvals_v.at[j], tsem)
           for j in range(NCHUNK)]
    sds = []
    s2ds = []
    for j in range(NCHUNK):
        gds[j].wait()
        d = pltpu.async_copy(rows_v.at[j % _NB], acc_s.at[dst_v.at[j]], s1sem, add=True)
        if j + _NB < NCHUNK:
            d.wait()
            gds[j + _NB] = pltpu.async_copy(
                g0_hbm.at[src_v.at[j + _NB]], rows_v.at[(j + _NB) % _NB], gsem)
        else:
            sds.append(d)
        tds[j].wait()
        s2ds.append(pltpu.async_copy(tvals_v.at[j], t_s.at[src_v.at[j]], s2sem, add=True))
    for d in sds:
        d.wait()
    for d in s2ds:
        d.wait()
    plsc.subcore_barrier()

    @pl.when(cid == 0)
    def _():
        pltpu.sync_copy(acc_s.at[pl.ds(sid * STRIPE, STRIPE), :],
                        acc0_out.at[pl.ds(sid * STRIPE, STRIPE), :])
        pltpu.sync_copy(t_s.at[pl.ds(sid * STRIPE, STRIPE)],
                        t0_out.at[pl.ds(sid * STRIPE, STRIPE)])

    @pl.when(cid == 1)
    def _():
        pltpu.sync_copy(acc_s.at[pl.ds(sid * STRIPE, STRIPE), :],
                        acc1_out.at[pl.ds(sid * STRIPE, STRIPE), :])
        pltpu.sync_copy(t_s.at[pl.ds(sid * STRIPE, STRIPE)],
                        t1_out.at[pl.ds(sid * STRIPE, STRIPE)])


_R2 = 5120


def _tc2_body(a0_ref, a1_ref, g0_ref, dinv_ref, t0_ref, t1_ref,
              b1_ref, w2_ref, b2_ref, w3_ref, b3_ref, out_ref, vacc):
    i = pl.program_id(0)
    dinv = dinv_ref[...]
    acc = a0_ref[...] + a1_ref[...] + g0_ref[...]
    h1 = jnp.maximum(dinv * acc + b1_ref[...], 0.0)
    t = (t0_ref[...] + t1_ref[...]).reshape(_R2, 1)
    w = dinv * (t + dinv)
    ridx = lax.broadcasted_iota(jnp.int32, (_R2, 1), 0) + i * _R2
    part = jnp.sum(jnp.where(ridx < N, h1 * w, 0.0), axis=0, keepdims=True)

    @pl.when(i == 0)
    def _():
        vacc[...] = part

    @pl.when(i == NP // _R2 - 1)
    def _():
        v = (vacc[...] + part) * (1.0 / N)
        pooled = jnp.dot(v, w2_ref[...], preferred_element_type=jnp.float32) + b2_ref[...]
        out_ref[...] = jnp.dot(pooled, w3_ref[...], preferred_element_type=jnp.float32) + b3_ref[...]


def _tc2(acc0, acc1, g0, dinv, t0, t1, b1, W2, b2, W3, b3):
    return pl.pallas_call(
        _tc2_body,
        grid=(NP // _R2,),
        in_specs=[
            pl.BlockSpec((_R2, H), lambda i: (i, 0)),
            pl.BlockSpec((_R2, H), lambda i: (i, 0)),
            pl.BlockSpec((_R2, H), lambda i: (i, 0)),
            pl.BlockSpec((_R2, 1), lambda i: (i, 0)),
            pl.BlockSpec((_R2,), lambda i: (i,)),
            pl.BlockSpec((_R2,), lambda i: (i,)),
            pl.BlockSpec((1, H), lambda i: (0, 0)),
            pl.BlockSpec((H, 40), lambda i: (0, 0)),
            pl.BlockSpec((1, 40), lambda i: (0, 0)),
            pl.BlockSpec((40, 3), lambda i: (0, 0)),
            pl.BlockSpec((1, 3), lambda i: (0, 0)),
        ],
        out_specs=pl.BlockSpec((1, 3), lambda i: (0, 0)),
        out_shape=jax.ShapeDtypeStruct((1, 3), jnp.float32),
        scratch_shapes=[pltpu.VMEM((1, H), jnp.float32)],
    )(acc0, acc1, g0, dinv, t0, t1, b1, W2, b2, W3, b3)


def kernel(x, edge_index, W1, b1, W2, b2, W3, b3):
    pad = (N + (jnp.arange(EP - E, dtype=jnp.int32) % 128)).astype(jnp.int32)
    srcp = jnp.concatenate([edge_index[0], pad]).reshape(NT, NCHUNK, CHUNK)
    dstp = jnp.concatenate([edge_index[1], pad]).reshape(NT, NCHUNK, CHUNK)
    z1 = jnp.zeros((STRIPE,), jnp.float32)
    zH = jnp.zeros((STRIPE, H), jnp.float32)

    deg0, deg1 = _sc_deg(dstp, z1)
    h0 = _tc1a(x, W1)
    g0, dinv1, dinv2 = _tc1b(h0, deg0, deg1)
    acc0, acc1, t0, t1 = _sc_edge(g0, dinv1, srcp, dstp, zH, z1)
    return _tc2(acc0, acc1, g0, dinv2, t0, t1,
                b1.reshape(1, H), W2, b2.reshape(1, 40), W3, b3.reshape(1, 3))


# speedup vs baseline: 43.4768x; 1.0232x over previous
import functools

import jax
import jax.numpy as jnp
from jax import lax
from jax.experimental import pallas as pl
from jax.experimental.pallas import tpu as pltpu, tpu_sc as plsc

N = 10000
D = 256
H = 16
E = 160000
NP = 10240
NT = 32
CHUNK = 128
NCHUNK = 40
EP = NT * NCHUNK * CHUNK
STRIPE = NP // 16

_mesh = plsc.VectorSubcoreMesh(core_axis_name="c", subcore_axis_name="s")


@functools.partial(
    pl.kernel,
    out_type=[
        jax.ShapeDtypeStruct((NP,), jnp.float32),
        jax.ShapeDtypeStruct((NP,), jnp.float32),
    ],
    mesh=_mesh,
    scratch_types=[
        pltpu.VMEM((NCHUNK, CHUNK), jnp.int32),
        pltpu.VMEM((CHUNK,), jnp.float32),
        pltpu.VMEM_SHARED((NP,), jnp.float32),
        pltpu.SemaphoreType.DMA,
    ],
)
def _sc_deg(dst_hbm, z1_hbm, deg0_out, deg1_out, idx_v, ones_v, deg_s, sem):
    cid = lax.axis_index("c")
    sid = lax.axis_index("s")
    wid = sid * 2 + cid
    pltpu.sync_copy(dst_hbm.at[wid], idx_v)
    for k in range(CHUNK // 16):
        ones_v[pl.ds(k * 16, 16)] = jnp.ones((16,), jnp.float32)
    pltpu.sync_copy(z1_hbm, deg_s.at[pl.ds(sid * STRIPE, STRIPE)])
    plsc.subcore_barrier()

    descs = [pltpu.async_copy(ones_v, deg_s.at[idx_v.at[j]], sem, add=True)
             for j in range(NCHUNK)]
    for d in descs:
        d.wait()
    plsc.subcore_barrier()

    @pl.when(cid == 0)
    def _():
        pltpu.sync_copy(deg_s.at[pl.ds(sid * STRIPE, STRIPE)],
                        deg0_out.at[pl.ds(sid * STRIPE, STRIPE)])

    @pl.when(cid == 1)
    def _():
        pltpu.sync_copy(deg_s.at[pl.ds(sid * STRIPE, STRIPE)],
                        deg1_out.at[pl.ds(sid * STRIPE, STRIPE)])


_R1 = 1024


def _tc1a_body(x_ref, w1_ref, h0t_ref):
    h0t_ref[...] = lax.dot_general(w1_ref[...], x_ref[...],
                                   (((0,), (1,)), ((), ())),
                                   preferred_element_type=jnp.float32)


def _tc1a(x, W1):
    return pl.pallas_call(
        _tc1a_body,
        grid=(NP // _R1,),
        in_specs=[
            pl.BlockSpec((_R1, D), lambda i: (i, 0)),
            pl.BlockSpec((D, H), lambda i: (0, 0)),
        ],
        out_specs=pl.BlockSpec((H, _R1), lambda i: (0, i)),
        out_shape=jax.ShapeDtypeStruct((H, NP), jnp.float32),
    )(x, W1)


def _tc1b_body(h0t_ref, d0_ref, d1_ref, g0_ref, dinv1_ref, dinv2_ref):
    deg = d0_ref[...] + d1_ref[...] + 1.0
    dinv = lax.rsqrt(deg)
    dinv1_ref[...] = dinv
    dinv2 = dinv.reshape(_R1, 1)
    dinv2_ref[...] = dinv2
    g0_ref[...] = h0t_ref[...].T * dinv2


def _tc1b(h0t, deg0, deg1):
    return pl.pallas_call(
        _tc1b_body,
        grid=(NP // _R1,),
        in_specs=[
            pl.BlockSpec((H, _R1), lambda i: (0, i)),
            pl.BlockSpec((_R1,), lambda i: (i,)),
            pl.BlockSpec((_R1,), lambda i: (i,)),
        ],
        out_specs=[
            pl.BlockSpec((_R1, H), lambda i: (i, 0)),
            pl.BlockSpec((_R1,), lambda i: (i,)),
            pl.BlockSpec((_R1, 1), lambda i: (i, 0)),
        ],
        out_shape=[
            jax.ShapeDtypeStruct((NP, H), jnp.float32),
            jax.ShapeDtypeStruct((NP,), jnp.float32),
            jax.ShapeDtypeStruct((NP, 1), jnp.float32),
        ],
    )(h0t, deg0, deg1)


_NB = 24


@functools.partial(
    pl.kernel,
    out_type=[
        jax.ShapeDtypeStruct((NP, H), jnp.float32),
        jax.ShapeDtypeStruct((NP, H), jnp.float32),
        jax.ShapeDtypeStruct((NP,), jnp.float32),
        jax.ShapeDtypeStruct((NP,), jnp.float32),
    ],
    mesh=_mesh,
    scratch_types=[
        pltpu.VMEM((NCHUNK, CHUNK), jnp.int32),
        pltpu.VMEM((NCHUNK, CHUNK), jnp.int32),
        pltpu.VMEM((_NB, CHUNK, H), jnp.float32),
        pltpu.VMEM((NCHUNK, CHUNK), jnp.float32),
        pltpu.VMEM_SHARED((NP, H), jnp.float32),
        pltpu.VMEM_SHARED((NP,), jnp.float32),
        pltpu.SemaphoreType.DMA,
        pltpu.SemaphoreType.DMA,
        pltpu.SemaphoreType.DMA,
        pltpu.SemaphoreType.DMA,
    ],
    compiler_params=pltpu.CompilerParams(use_tc_tiling_on_sc=False),
)
def _sc_edge(g0_hbm, dinv_hbm, src_hbm, dst_hbm, zH_hbm, z1_hbm,
             acc0_out, acc1_out, t0_out, t1_out,
             src_v, dst_v, rows_v, tvals_v, acc_s, t_s,
             gsem, tsem, s1sem, s2sem):
    cid = lax.axis_index("c")
    sid = lax.axis_index("s")
    wid = sid * 2 + cid
    pltpu.sync_copy(src_hbm.at[wid], src_v)
    pltpu.sync_copy(dst_hbm.at[wid], dst_v)
    pltpu.sync_copy(zH_hbm, acc_s.at[pl.ds(sid * STRIPE, STRIPE), :])
    pltpu.sync_copy(z1_hbm, t_s.at[pl.ds(sid * STRIPE, STRIPE)])
    plsc.subcore_barrier()

    gds = {}
    for j in range(_NB):
        gds[j] = pltpu.async_copy(g0_hbm.at[src_v.at[j]], rows_v.at[j], gsem)
    tds = [pltpu.async_copy(dinv_hbm.at[dst_v.at[j]], tvals_v.at[j], tsem)
           for j in range(NCHUNK)]
    sds = []
    s2ds = []
    for j in range(NCHUNK):
        gds[j].wait()
        d = pltpu.async_copy(rows_v.at[j % _NB], acc_s.at[dst_v.at[j]], s1sem, add=True)
        if j + _NB < NCHUNK:
            d.wait()
            gds[j + _NB] = pltpu.async_copy(
                g0_hbm.at[src_v.at[j + _NB]], rows_v.at[(j + _NB) % _NB], gsem)
        else:
            sds.append(d)
        tds[j].wait()
        s2ds.append(pltpu.async_copy(tvals_v.at[j], t_s.at[src_v.at[j]], s2sem, add=True))
    for d in sds:
        d.wait()
    for d in s2ds:
        d.wait()
    plsc.subcore_barrier()

    @pl.when(cid == 0)
    def _():
        pltpu.sync_copy(acc_s.at[pl.ds(sid * STRIPE, STRIPE), :],
                        acc0_out.at[pl.ds(sid * STRIPE, STRIPE), :])
        pltpu.sync_copy(t_s.at[pl.ds(sid * STRIPE, STRIPE)],
                        t0_out.at[pl.ds(sid * STRIPE, STRIPE)])

    @pl.when(cid == 1)
    def _():
        pltpu.sync_copy(acc_s.at[pl.ds(sid * STRIPE, STRIPE), :],
                        acc1_out.at[pl.ds(sid * STRIPE, STRIPE), :])
        pltpu.sync_copy(t_s.at[pl.ds(sid * STRIPE, STRIPE)],
                        t1_out.at[pl.ds(sid * STRIPE, STRIPE)])


_R2 = 5120


def _tc2_body(a0_ref, a1_ref, g0_ref, dinv_ref, t0_ref, t1_ref,
              b1_ref, w2_ref, b2_ref, w3_ref, b3_ref, out_ref, vacc):
    i = pl.program_id(0)
    dinv = dinv_ref[...]
    acc = a0_ref[...] + a1_ref[...] + g0_ref[...]
    h1 = jnp.maximum(dinv * acc + b1_ref[...], 0.0)
    t = (t0_ref[...] + t1_ref[...]).reshape(_R2, 1)
    w = dinv * (t + dinv)
    ridx = lax.broadcasted_iota(jnp.int32, (_R2, 1), 0) + i * _R2
    part = jnp.sum(jnp.where(ridx < N, h1 * w, 0.0), axis=0, keepdims=True)

    @pl.when(i == 0)
    def _():
        vacc[...] = part

    @pl.when(i == NP // _R2 - 1)
    def _():
        v = (vacc[...] + part) * (1.0 / N)
        pooled = jnp.dot(v, w2_ref[...], preferred_element_type=jnp.float32) + b2_ref[...]
        out_ref[...] = jnp.dot(pooled, w3_ref[...], preferred_element_type=jnp.float32) + b3_ref[...]


def _tc2(acc0, acc1, g0, dinv, t0, t1, b1, W2, b2, W3, b3):
    return pl.pallas_call(
        _tc2_body,
        grid=(NP // _R2,),
        in_specs=[
            pl.BlockSpec((_R2, H), lambda i: (i, 0)),
            pl.BlockSpec((_R2, H), lambda i: (i, 0)),
            pl.BlockSpec((_R2, H), lambda i: (i, 0)),
            pl.BlockSpec((_R2, 1), lambda i: (i, 0)),
            pl.BlockSpec((_R2,), lambda i: (i,)),
            pl.BlockSpec((_R2,), lambda i: (i,)),
            pl.BlockSpec((1, H), lambda i: (0, 0)),
            pl.BlockSpec((H, 40), lambda i: (0, 0)),
            pl.BlockSpec((1, 40), lambda i: (0, 0)),
            pl.BlockSpec((40, 3), lambda i: (0, 0)),
            pl.BlockSpec((1, 3), lambda i: (0, 0)),
        ],
        out_specs=pl.BlockSpec((1, 3), lambda i: (0, 0)),
        out_shape=jax.ShapeDtypeStruct((1, 3), jnp.float32),
        scratch_shapes=[pltpu.VMEM((1, H), jnp.float32)],
    )(acc0, acc1, g0, dinv, t0, t1, b1, W2, b2, W3, b3)


def kernel(x, edge_index, W1, b1, W2, b2, W3, b3):
    pad = (N + (jnp.arange(EP - E, dtype=jnp.int32) % 128)).astype(jnp.int32)
    srcp = jnp.concatenate([edge_index[0], pad]).reshape(NT, NCHUNK, CHUNK)
    dstp = jnp.concatenate([edge_index[1], pad]).reshape(NT, NCHUNK, CHUNK)
    z1 = jnp.zeros((STRIPE,), jnp.float32)
    zH = jnp.zeros((STRIPE, H), jnp.float32)

    deg0, deg1 = _sc_deg(dstp, z1)
    h0t = _tc1a(x, W1)
    g0, dinv1, dinv2 = _tc1b(h0t, deg0, deg1)
    acc0, acc1, t0, t1 = _sc_edge(g0, dinv1, srcp, dstp, zH, z1)
    return _tc2(acc0, acc1, g0, dinv2, t0, t1,
                b1.reshape(1, H), W2, b2.reshape(1, 40), W3, b3.reshape(1, 3))
